# baseline (speedup 1.0000x reference)
import functools

import jax
import jax.numpy as jnp
from jax import lax
from jax.experimental import pallas as pl
from jax.experimental.pallas import tpu as pltpu
from jax.experimental.pallas import tpu_sc as plsc

_NC = 2
_NS = 16
_NW = _NC * _NS
_L = 200
_RB = 32
_NVEC = 13
_CHI = _RB * _L
_CHO = _RB * _L * 4


def _dg(vec, idx):
    return vec.at[idx].get(mode="promise_in_bounds")


def _body(idx_hbm, par_hbm, out_hbm, par_v,
          idx_v0, idx_v1, out_v0, out_v1, s_i0, s_i1, s_o0, s_o1):
    wid = lax.axis_index("s") * _NC + lax.axis_index("c")
    rows_per_w = idx_hbm.shape[0] // (_L * _NW)
    npair = rows_per_w // (2 * _RB)

    pltpu.sync_copy(par_hbm, par_v)
    lane = lax.iota(jnp.int32, 16)
    emb_v = par_v[pl.ds(0, 16)]
    w_v = par_v[pl.ds(16, 16)]
    b_v = par_v[pl.ds(32, 16)]
    k2 = jnp.minimum(lane, 7) * 2
    luts = []
    for c in range(4):
        cc = jnp.full((16,), c, jnp.int32)
        luts.append(_dg(emb_v, k2) * _dg(w_v, cc)
                    + _dg(emb_v, k2 + 1) * _dg(w_v, cc + 4)
                    + _dg(b_v, cc))

    ibase = wid * rows_per_w * _L
    obase = wid * rows_per_w * _L * 4
    bufs = ((idx_v0, out_v0, s_i0, s_o0), (idx_v1, out_v1, s_i1, s_o1))

    def istart(blk, half):
        iv, _, sem, _ = bufs[half]
        return pltpu.async_copy(
            idx_hbm.at[pl.ds(ibase + blk * _CHI, _CHI)],
            iv.at[pl.ds(0, _CHI)], sem)

    def iwait(half):
        iv, _, sem, _ = bufs[half]
        pltpu.make_async_copy(idx_hbm.at[pl.ds(0, _CHI)],
                              iv.at[pl.ds(0, _CHI)], sem).wait()

    def ostart(blk, half):
        _, ov, _, sem = bufs[half]
        return pltpu.async_copy(
            ov.at[pl.ds(0, _CHO)],
            out_hbm.at[pl.ds(obase + blk * _CHO, _CHO)], sem)

    def owait(half):
        _, ov, _, sem = bufs[half]
        pltpu.make_async_copy(ov.at[pl.ds(0, _CHO)],
                              out_hbm.at[pl.ds(0, _CHO)], sem).wait()

    def compute(half):
        iv, ov, _, _ = bufs[half]

        def row_body(r, carry):
            for c in range(4):
                for j in range(_NVEC):
                    v = iv[pl.ds(r * _L + j * 16, 16)]
                    ov[pl.ds(r * _L * 4 + c * _L + j * 16, 16)] = \
                        _dg(luts[c], v)
            return carry

        lax.fori_loop(0, _RB, row_body, 0)

    istart(0, 0)
    istart(1, 1)

    def pair_body(t, carry):
        blk = t * 2
        iwait(0)

        @pl.when(t > 0)
        def _():
            owait(0)

        compute(0)
        ostart(blk, 0)

        @pl.when(t + 1 < npair)
        def _():
            istart(blk + 2, 0)

        iwait(1)

        @pl.when(t > 0)
        def _():
            owait(1)

        compute(1)
        ostart(blk + 1, 1)

        @pl.when(t + 1 < npair)
        def _():
            istart(blk + 3, 1)

        return carry

    lax.fori_loop(0, npair, pair_body, 0)
    owait(0)
    owait(1)


def kernel(inputs, emb_table, W, b):
    B, L = inputs.shape
    N = B * L
    idx_flat = inputs.reshape(N).astype(jnp.int32)
    par = jnp.zeros((48,), jnp.float32)
    par = par.at[0:6].set(emb_table.reshape(-1))
    par = par.at[16:24].set(W.reshape(-1))
    par = par.at[32:36].set(b)

    mesh = plsc.VectorSubcoreMesh(core_axis_name="c", subcore_axis_name="s")
    run = functools.partial(
        pl.kernel,
        mesh=mesh,
        out_type=jax.ShapeDtypeStruct((N * 4,), jnp.float32),
        scratch_types=[
            pltpu.VMEM((48,), jnp.float32),
            pltpu.VMEM((_CHI + 8,), jnp.int32),
            pltpu.VMEM((_CHI + 8,), jnp.int32),
            pltpu.VMEM((_CHO + 8,), jnp.float32),
            pltpu.VMEM((_CHO + 8,), jnp.float32),
            pltpu.SemaphoreType.DMA,
            pltpu.SemaphoreType.DMA,
            pltpu.SemaphoreType.DMA,
            pltpu.SemaphoreType.DMA,
        ],
    )(_body)
    out = run(idx_flat, par)
    return jnp.swapaxes(out.reshape(B, 4, L), 1, 2)


# speedup vs baseline: 1.2875x; 1.2875x over previous
import functools

import jax
import jax.numpy as jnp
from jax import lax
from jax.experimental import pallas as pl
from jax.experimental.pallas import tpu as pltpu
from jax.experimental.pallas import tpu_sc as plsc

_NC = 2
_NS = 16
_NW = _NC * _NS
_L = 200
_RB = 32
_CHI = _RB * _L
_CHO = _RB * _L * 4


def _dg(vec, idx):
    return vec.at[idx].get(mode="promise_in_bounds")


def _body(idx_hbm, par_hbm, out_hbm, par_v,
          idx_v0, idx_v1, out_v0, out_v1, s_i0, s_i1, s_o0, s_o1):
    wid = lax.axis_index("s") * _NC + lax.axis_index("c")
    rows_per_w = idx_hbm.shape[0] // (_L * _NW)
    npair = rows_per_w // (2 * _RB)

    pltpu.sync_copy(par_hbm, par_v)
    lane = lax.iota(jnp.int32, 16)
    emb_v = par_v[pl.ds(0, 16)]
    w_v = par_v[pl.ds(16, 16)]
    b_v = par_v[pl.ds(32, 16)]
    k2 = jnp.minimum(lane, 7) * 2
    luts = []
    for c in range(4):
        cc = jnp.full((16,), c, jnp.int32)
        luts.append(_dg(emb_v, k2) * _dg(w_v, cc)
                    + _dg(emb_v, k2 + 1) * _dg(w_v, cc + 4)
                    + _dg(b_v, cc))

    ibase = wid * rows_per_w * _L
    obase = wid * rows_per_w * _L * 4
    bufs = ((idx_v0, out_v0, s_i0, s_o0), (idx_v1, out_v1, s_i1, s_o1))

    def istart(blk, half):
        iv, _, sem, _ = bufs[half]
        return pltpu.async_copy(
            idx_hbm.at[pl.ds(ibase + blk * _CHI, _CHI)],
            iv.at[pl.ds(0, _CHI)], sem)

    def iwait(half):
        iv, _, sem, _ = bufs[half]
        pltpu.make_async_copy(idx_hbm.at[pl.ds(0, _CHI)],
                              iv.at[pl.ds(0, _CHI)], sem).wait()

    def ostart(blk, half):
        _, ov, _, sem = bufs[half]
        return pltpu.async_copy(
            ov.at[pl.ds(0, _CHO)],
            out_hbm.at[pl.ds(obase + blk * _CHO, _CHO)], sem)

    def owait(half):
        _, ov, _, sem = bufs[half]
        pltpu.make_async_copy(ov.at[pl.ds(0, _CHO)],
                              out_hbm.at[pl.ds(0, _CHO)], sem).wait()

    def compute(half):
        iv, ov, _, _ = bufs[half]

        def row_body(r, carry):
            ib = r * _L
            ob = r * _L * 4

            def quad(k, c2):
                off = k * 64
                for jj in range(4):
                    o = off + jj * 16
                    v = iv[pl.ds(ib + o, 16)]
                    for c in range(4):
                        ov[pl.ds(ob + c * _L + o, 16)] = _dg(luts[c], v)
                return c2

            lax.fori_loop(0, 3, quad, 0)
            v = iv[pl.ds(ib + _L - 16, 16)]
            for c in range(4):
                ov[pl.ds(ob + c * _L + _L - 16, 16)] = _dg(luts[c], v)
            return carry

        lax.fori_loop(0, _RB, row_body, 0)

    istart(0, 0)
    istart(1, 1)

    def pair_body(t, carry):
        blk = t * 2
        iwait(0)

        @pl.when(t > 0)
        def _():
            owait(0)

        compute(0)
        ostart(blk, 0)

        @pl.when(t + 1 < npair)
        def _():
            istart(blk + 2, 0)

        iwait(1)

        @pl.when(t > 0)
        def _():
            owait(1)

        compute(1)
        ostart(blk + 1, 1)

        @pl.when(t + 1 < npair)
        def _():
            istart(blk + 3, 1)

        return carry

    lax.fori_loop(0, npair, pair_body, 0)
    owait(0)
    owait(1)


def kernel(inputs, emb_table, W, b):
    B, L = inputs.shape
    N = B * L
    idx_flat = inputs.reshape(N).astype(jnp.int32)
    par = jnp.zeros((48,), jnp.float32)
    par = par.at[0:6].set(emb_table.reshape(-1))
    par = par.at[16:24].set(W.reshape(-1))
    par = par.at[32:36].set(b)

    mesh = plsc.VectorSubcoreMesh(core_axis_name="c", subcore_axis_name="s")
    run = functools.partial(
        pl.kernel,
        mesh=mesh,
        out_type=jax.ShapeDtypeStruct((N * 4,), jnp.float32),
        scratch_types=[
            pltpu.VMEM((48,), jnp.float32),
            pltpu.VMEM((_CHI + 8,), jnp.int32),
            pltpu.VMEM((_CHI + 8,), jnp.int32),
            pltpu.VMEM((_CHO + 8,), jnp.float32),
            pltpu.VMEM((_CHO + 8,), jnp.float32),
            pltpu.SemaphoreType.DMA,
            pltpu.SemaphoreType.DMA,
            pltpu.SemaphoreType.DMA,
            pltpu.SemaphoreType.DMA,
        ],
    )(_body)
    out = run(idx_flat, par)
    return jnp.swapaxes(out.reshape(B, 4, L), 1, 2)
